# baseline (speedup 1.0000x reference)
import functools

import jax
import jax.numpy as jnp
from jax import lax
from jax.experimental import pallas as pl
from jax.experimental.pallas import tpu as pltpu
from jax.experimental.pallas import tpu_sc as plsc

_NC = 2
_NS = 16
_NW = _NC * _NS
_L = 16

_CHUNK = 4096
_IDX_ROW = 128
_GPC = _CHUNK // _IDX_ROW
_D = 16

_PAD_WORKERS = 25
_PAD_ROWS = 4000


def _pad_body(n_rows, src_len, table9_hbm, table16_hbm, in_v, out_v):
    wid = lax.axis_index("s") * _NC + lax.axis_index("c")
    iota = lax.iota(jnp.int32, _L)

    @pl.when(wid < _PAD_WORKERS)
    def _():
        row0 = wid * _PAD_ROWS
        src_off = pl.multiple_of(row0 * 9, 8)
        n_in = jnp.minimum(_PAD_ROWS * 9 + 8, src_len - src_off)
        pltpu.sync_copy(table9_hbm.at[pl.ds(src_off, _PAD_ROWS * 9)],
                        in_v.at[pl.ds(0, _PAD_ROWS * 9)])
        zeros = jnp.zeros((_L,), jnp.float32)
        del n_in

        def group(g, _):
            lanes = jnp.broadcast_to(g * _L, (_L,)).astype(jnp.int32) + iota
            src = lanes * 9
            for e in range(9):
                col = plsc.load_gather(in_v, [src + e])
                plsc.store_scatter(out_v, [lanes, jnp.full((_L,), e, jnp.int32)], col)
            for e in range(9, _D):
                plsc.store_scatter(out_v, [lanes, jnp.full((_L,), e, jnp.int32)], zeros)
            return ()

        lax.fori_loop(0, _PAD_ROWS // _L, group, ())
        pltpu.sync_copy(out_v, table16_hbm.at[pl.ds(row0, _PAD_ROWS)])


@jax.jit
def _pad_call(table9):
    n_rows = table9.shape[0] // 9
    grid_kernel = pl.kernel(
        functools.partial(_pad_body, n_rows, table9.shape[0]),
        out_type=jax.ShapeDtypeStruct((n_rows, _D), jnp.float32),
        mesh=plsc.VectorSubcoreMesh(core_axis_name="c", subcore_axis_name="s"),
        compiler_params=pltpu.CompilerParams(
            needs_layout_passes=False, use_tc_tiling_on_sc=False),
        scratch_types=[
            pltpu.VMEM((_PAD_ROWS * 9,), jnp.float32),
            pltpu.VMEM((_PAD_ROWS, _D), jnp.float32),
        ],
    )
    return grid_kernel(table9)


def _texture_body(n_tokens, hw, w_len, table_hbm, idx_hbm, bary_hbm, out_hbm,
                  idx_v, bary_v, rows_v, out_v, sem):
    per_worker = n_tokens // _NW
    n_chunks = per_worker // _CHUNK
    row3 = 3 * w_len
    wid = lax.axis_index("s") * _NC + lax.axis_index("c")
    iota = lax.iota(jnp.int32, _L)

    def chunk_body(chunk, _):
        tok_base = pl.multiple_of(wid * per_worker + chunk * _CHUNK, _CHUNK)
        pltpu.sync_copy(idx_hbm.at[pl.ds(tok_base, _CHUNK)], idx_v)
        pltpu.sync_copy(bary_hbm.at[pl.ds(3 * tok_base, 3 * _CHUNK)], bary_v)
        copies = [
            pltpu.async_copy(
                table_hbm.at[idx_v.at[pl.ds(j * _IDX_ROW, _IDX_ROW)]],
                rows_v.at[pl.ds(j * _IDX_ROW, _IDX_ROW)],
                sem,
            )
            for j in range(_GPC)
        ]
        for cp in copies:
            cp.wait()

        def group_body(j, _):
            base = j * _IDX_ROW
            for k in range(_IDX_ROW // _L):
                g16 = base + k * _L
                lane = jnp.broadcast_to(g16, (_L,)).astype(jnp.int32) + iota
                idx16 = idx_v[pl.ds(g16, _L)]
                m = idx16 > 0
                h_local = g16 // w_len
                boff = h_local * row3 + (g16 - h_local * w_len)
                bw = [bary_v[pl.ds(boff + v * w_len, _L)] for v in range(3)]
                ge = [plsc.load_gather(rows_v, [lane, jnp.full((_L,), e, jnp.int32)])
                      for e in range(9)]
                for c in range(3):
                    oc = bw[0] * ge[c] + bw[1] * ge[3 + c] + bw[2] * ge[6 + c]
                    oc = jnp.where(m, oc, 0.0)
                    out_v[pl.ds(c * _CHUNK + g16, _L)] = oc
            return ()

        lax.fori_loop(0, _GPC, group_body, ())

        b_img = tok_base // hw
        p_base = tok_base - b_img * hw
        for c in range(3):
            off = pl.multiple_of(b_img * 3 * hw + c * hw + p_base, _CHUNK)
            pltpu.sync_copy(out_v.at[pl.ds(c * _CHUNK, _CHUNK)],
                            out_hbm.at[pl.ds(off, _CHUNK)])
        return ()

    lax.fori_loop(0, n_chunks, chunk_body, ())


@functools.partial(jax.jit, static_argnums=(3, 4, 5))
def _texture_call(table, idx, bary, n_tokens, hw, w_len):
    grid_kernel = pl.kernel(
        functools.partial(_texture_body, n_tokens, hw, w_len),
        out_type=jax.ShapeDtypeStruct((3 * n_tokens,), jnp.float32),
        mesh=plsc.VectorSubcoreMesh(core_axis_name="c", subcore_axis_name="s"),
        compiler_params=pltpu.CompilerParams(
            needs_layout_passes=False, use_tc_tiling_on_sc=False),
        scratch_types=[
            pltpu.VMEM((_CHUNK,), jnp.int32),
            pltpu.VMEM((3 * _CHUNK,), jnp.float32),
            pltpu.VMEM((_CHUNK, _D), jnp.float32),
            pltpu.VMEM((3 * _CHUNK,), jnp.float32),
            pltpu.SemaphoreType.DMA,
        ],
    )
    return grid_kernel(table, idx, bary)


def kernel(pix_to_face, bary_coords, face_verts_colors):
    B, H, W, K = pix_to_face.shape
    F, V, C = face_verts_colors.shape
    n = B * H * W * K
    hw = H * W * K
    idx = pix_to_face.reshape(n).astype(jnp.int32)
    bary = (jnp.transpose(bary_coords.reshape(B, H, W * K, 3), (0, 1, 3, 2))
            .reshape(B, H, 3 * W * K).reshape(3 * n))
    table16 = _pad_call(face_verts_colors.reshape(F * V * C))
    out_flat = _texture_call(table16, idx, bary, n, hw, W * K)
    return out_flat.reshape(B, C, H, W)


# speedup vs baseline: 2.0016x; 2.0016x over previous
import functools

import jax
import jax.numpy as jnp
from jax import lax
from jax.experimental import pallas as pl
from jax.experimental.pallas import tpu as pltpu
from jax.experimental.pallas import tpu_sc as plsc

_NC = 2
_NS = 16
_NW = _NC * _NS
_L = 16

_CHUNK = 4096
_IDX_ROW = 128
_GPC = _CHUNK // _IDX_ROW
_D = 16

_PAD_WORKERS = 25
_PAD_ROWS = 4000


def _pad_body(n_rows, table9_hbm, table16_hbm, in_v, out_v):
    wid = lax.axis_index("s") * _NC + lax.axis_index("c")
    iota = lax.iota(jnp.int32, _L)

    @pl.when(wid < _PAD_WORKERS)
    def _():
        row0 = wid * _PAD_ROWS
        for e in range(9):
            pltpu.sync_copy(
                table9_hbm.at[pl.ds(e * n_rows + row0, _PAD_ROWS)],
                in_v.at[pl.ds(e * _PAD_ROWS, _PAD_ROWS)])
        zeros = jnp.zeros((_L,), jnp.float32)

        def group(g, _):
            lanes = jnp.broadcast_to(g * _L, (_L,)).astype(jnp.int32) + iota
            for e in range(9):
                col = in_v[pl.ds(e * _PAD_ROWS + g * _L, _L)]
                plsc.store_scatter(out_v, [lanes, jnp.full((_L,), e, jnp.int32)], col)
            for e in range(9, _D):
                plsc.store_scatter(out_v, [lanes, jnp.full((_L,), e, jnp.int32)], zeros)
            return ()

        lax.fori_loop(0, _PAD_ROWS // _L, group, ())
        pltpu.sync_copy(out_v, table16_hbm.at[pl.ds(row0, _PAD_ROWS)])


@jax.jit
def _pad_call(table9):
    n_rows = table9.shape[0] // 9
    grid_kernel = pl.kernel(
        functools.partial(_pad_body, n_rows),
        out_type=jax.ShapeDtypeStruct((n_rows, _D), jnp.float32),
        mesh=plsc.VectorSubcoreMesh(core_axis_name="c", subcore_axis_name="s"),
        compiler_params=pltpu.CompilerParams(
            needs_layout_passes=False, use_tc_tiling_on_sc=False),
        scratch_types=[
            pltpu.VMEM((_PAD_ROWS * 9,), jnp.float32),
            pltpu.VMEM((_PAD_ROWS, _D), jnp.float32),
        ],
    )
    return grid_kernel(table9)


def _texture_body(n_tokens, hw, w_len, table_hbm, idx_hbm, bary_hbm, out_hbm,
                  idx_v, bary_v, rows_v, out_v, sem):
    per_worker = n_tokens // _NW
    n_chunks = per_worker // _CHUNK
    row3 = 3 * w_len
    wid = lax.axis_index("s") * _NC + lax.axis_index("c")
    iota = lax.iota(jnp.int32, _L)

    def chunk_body(chunk, _):
        tok_base = pl.multiple_of(wid * per_worker + chunk * _CHUNK, _CHUNK)
        pltpu.sync_copy(idx_hbm.at[pl.ds(tok_base, _CHUNK)], idx_v)
        pltpu.sync_copy(bary_hbm.at[pl.ds(3 * tok_base, 3 * _CHUNK)], bary_v)
        copies = [
            pltpu.async_copy(
                table_hbm.at[idx_v.at[pl.ds(j * _IDX_ROW, _IDX_ROW)]],
                rows_v.at[pl.ds(j * _IDX_ROW, _IDX_ROW)],
                sem,
            )
            for j in range(_GPC)
        ]
        for cp in copies:
            cp.wait()

        def group_body(j, _):
            base = j * _IDX_ROW
            for k in range(_IDX_ROW // _L):
                g16 = base + k * _L
                lane = jnp.broadcast_to(g16, (_L,)).astype(jnp.int32) + iota
                idx16 = idx_v[pl.ds(g16, _L)]
                m = idx16 > 0
                h_local = g16 // w_len
                boff = h_local * row3 + (g16 - h_local * w_len)
                bw = [bary_v[pl.ds(boff + v * w_len, _L)] for v in range(3)]
                ge = [plsc.load_gather(rows_v, [lane, jnp.full((_L,), e, jnp.int32)])
                      for e in range(9)]
                for c in range(3):
                    oc = bw[0] * ge[c] + bw[1] * ge[3 + c] + bw[2] * ge[6 + c]
                    oc = jnp.where(m, oc, 0.0)
                    out_v[pl.ds(c * _CHUNK + g16, _L)] = oc
            return ()

        lax.fori_loop(0, _GPC, group_body, ())

        b_img = tok_base // hw
        p_base = tok_base - b_img * hw
        for c in range(3):
            off = pl.multiple_of(b_img * 3 * hw + c * hw + p_base, _CHUNK)
            pltpu.sync_copy(out_v.at[pl.ds(c * _CHUNK, _CHUNK)],
                            out_hbm.at[pl.ds(off, _CHUNK)])
        return ()

    lax.fori_loop(0, n_chunks, chunk_body, ())


@functools.partial(jax.jit, static_argnums=(3, 4, 5))
def _texture_call(table, idx, bary, n_tokens, hw, w_len):
    grid_kernel = pl.kernel(
        functools.partial(_texture_body, n_tokens, hw, w_len),
        out_type=jax.ShapeDtypeStruct((3 * n_tokens,), jnp.float32),
        mesh=plsc.VectorSubcoreMesh(core_axis_name="c", subcore_axis_name="s"),
        compiler_params=pltpu.CompilerParams(
            needs_layout_passes=False, use_tc_tiling_on_sc=False),
        scratch_types=[
            pltpu.VMEM((_CHUNK,), jnp.int32),
            pltpu.VMEM((3 * _CHUNK,), jnp.float32),
            pltpu.VMEM((_CHUNK, _D), jnp.float32),
            pltpu.VMEM((3 * _CHUNK,), jnp.float32),
            pltpu.SemaphoreType.DMA,
        ],
    )
    return grid_kernel(table, idx, bary)


def kernel(pix_to_face, bary_coords, face_verts_colors):
    B, H, W, K = pix_to_face.shape
    F, V, C = face_verts_colors.shape
    n = B * H * W * K
    hw = H * W * K
    idx = pix_to_face.reshape(n).astype(jnp.int32)
    bary = jnp.transpose(bary_coords, (0, 1, 4, 3, 2)).reshape(3 * n)
    table16 = _pad_call(jnp.transpose(face_verts_colors, (1, 2, 0)).reshape(F * V * C))
    out_flat = _texture_call(table16, idx, bary, n, hw, W * K)
    return out_flat.reshape(B, C, H, W)


# speedup vs baseline: 2.7507x; 1.3743x over previous
import functools

import jax
import jax.numpy as jnp
from jax import lax
from jax.experimental import pallas as pl
from jax.experimental.pallas import tpu as pltpu
from jax.experimental.pallas import tpu_sc as plsc

_NC = 2
_NS = 16
_NW = _NC * _NS
_L = 16

_CHUNK = 2048
_IDX_ROW = 128
_GPC = _CHUNK // _IDX_ROW
_D = 16
_NBUF = 2

_PAD_WORKERS = 25
_PAD_ROWS = 4000


def _pad_body(n_rows, table9_hbm, table16_hbm, in_v, out_v):
    wid = lax.axis_index("s") * _NC + lax.axis_index("c")
    iota = lax.iota(jnp.int32, _L)

    @pl.when(wid < _PAD_WORKERS)
    def _():
        row0 = wid * _PAD_ROWS
        for e in range(9):
            pltpu.sync_copy(
                table9_hbm.at[pl.ds(e * n_rows + row0, _PAD_ROWS)],
                in_v.at[pl.ds(e * _PAD_ROWS, _PAD_ROWS)])
        zeros = jnp.zeros((_L,), jnp.float32)

        def group(g, _):
            lanes = jnp.broadcast_to(g * _L, (_L,)).astype(jnp.int32) + iota
            for e in range(9):
                col = in_v[pl.ds(e * _PAD_ROWS + g * _L, _L)]
                plsc.store_scatter(out_v, [lanes, jnp.full((_L,), e, jnp.int32)], col)
            for e in range(9, _D):
                plsc.store_scatter(out_v, [lanes, jnp.full((_L,), e, jnp.int32)], zeros)
            return ()

        lax.fori_loop(0, _PAD_ROWS // _L, group, ())
        pltpu.sync_copy(out_v, table16_hbm.at[pl.ds(row0, _PAD_ROWS)])


@jax.jit
def _pad_call(table9):
    n_rows = table9.shape[0] // 9
    grid_kernel = pl.kernel(
        functools.partial(_pad_body, n_rows),
        out_type=jax.ShapeDtypeStruct((n_rows, _D), jnp.float32),
        mesh=plsc.VectorSubcoreMesh(core_axis_name="c", subcore_axis_name="s"),
        compiler_params=pltpu.CompilerParams(
            needs_layout_passes=False, use_tc_tiling_on_sc=False),
        scratch_types=[
            pltpu.VMEM((_PAD_ROWS * 9,), jnp.float32),
            pltpu.VMEM((_PAD_ROWS, _D), jnp.float32),
        ],
    )
    return grid_kernel(table9)


def _texture_body(n_tokens, hw, w_len, table_hbm, idx_hbm, bary_hbm, out_hbm,
                  idx_v, bary_v, rows_v, out_v, sem_g0, sem_g1, sem_b0, sem_b1):
    per_worker = n_tokens // _NW
    n_chunks = per_worker // _CHUNK
    row3 = 3 * w_len
    wid = lax.axis_index("s") * _NC + lax.axis_index("c")
    iota = lax.iota(jnp.int32, _L)
    sem_g = [sem_g0, sem_g1]
    sem_b = [sem_b0, sem_b1]

    def stage(chunk, buf):
        tok_base = pl.multiple_of(wid * per_worker + chunk * _CHUNK, _CHUNK)
        pltpu.sync_copy(idx_hbm.at[pl.ds(tok_base, _CHUNK)], idx_v.at[buf])
        for j in range(_GPC):
            pltpu.async_copy(
                table_hbm.at[idx_v.at[buf].at[pl.ds(j * _IDX_ROW, _IDX_ROW)]],
                rows_v.at[buf].at[pl.ds(j * _IDX_ROW, _IDX_ROW)],
                sem_g[buf],
            )
        pltpu.async_copy(bary_hbm.at[pl.ds(3 * tok_base, 3 * _CHUNK)],
                         bary_v.at[buf], sem_b[buf])

    def drain(buf):
        for j in range(_GPC):
            pltpu.make_async_copy(
                table_hbm.at[idx_v.at[buf].at[pl.ds(j * _IDX_ROW, _IDX_ROW)]],
                rows_v.at[buf].at[pl.ds(j * _IDX_ROW, _IDX_ROW)],
                sem_g[buf],
            ).wait()
        pltpu.make_async_copy(bary_hbm.at[pl.ds(0, 3 * _CHUNK)],
                              bary_v.at[buf], sem_b[buf]).wait()

    def compute(chunk, buf):
        tok_base = pl.multiple_of(wid * per_worker + chunk * _CHUNK, _CHUNK)
        idx_b, bary_b, rows_b, out_b = (idx_v.at[buf], bary_v.at[buf],
                                        rows_v.at[buf], out_v.at[buf])

        def group_body(j, _):
            base = j * _IDX_ROW
            for k in range(_IDX_ROW // _L):
                g16 = base + k * _L
                lane = jnp.broadcast_to(g16, (_L,)).astype(jnp.int32) + iota
                idx16 = idx_b[pl.ds(g16, _L)]
                m = idx16 > 0
                h_local = g16 // w_len
                boff = h_local * row3 + (g16 - h_local * w_len)
                bw = [bary_b[pl.ds(boff + v * w_len, _L)] for v in range(3)]
                ge = [plsc.load_gather(rows_b, [lane, jnp.full((_L,), e, jnp.int32)])
                      for e in range(9)]
                for c in range(3):
                    oc = bw[0] * ge[c] + bw[1] * ge[3 + c] + bw[2] * ge[6 + c]
                    oc = jnp.where(m, oc, 0.0)
                    out_b[pl.ds(c * _CHUNK + g16, _L)] = oc
            return ()

        lax.fori_loop(0, _GPC, group_body, ())

        b_img = tok_base // hw
        p_base = tok_base - b_img * hw
        for c in range(3):
            off = pl.multiple_of(b_img * 3 * hw + c * hw + p_base, _CHUNK)
            pltpu.sync_copy(out_b.at[pl.ds(c * _CHUNK, _CHUNK)],
                            out_hbm.at[pl.ds(off, _CHUNK)])

    stage(0, 0)

    def pair_body(p, _):
        for buf in range(_NBUF):
            chunk = p * _NBUF + buf
            nxt = chunk + 1

            @pl.when(nxt < n_chunks)
            def _():
                stage(nxt, 1 - buf)

            drain(buf)
            compute(chunk, buf)
        return ()

    lax.fori_loop(0, n_chunks // _NBUF, pair_body, ())


@functools.partial(jax.jit, static_argnums=(3, 4, 5))
def _texture_call(table, idx, bary, n_tokens, hw, w_len):
    grid_kernel = pl.kernel(
        functools.partial(_texture_body, n_tokens, hw, w_len),
        out_type=jax.ShapeDtypeStruct((3 * n_tokens,), jnp.float32),
        mesh=plsc.VectorSubcoreMesh(core_axis_name="c", subcore_axis_name="s"),
        compiler_params=pltpu.CompilerParams(
            needs_layout_passes=False, use_tc_tiling_on_sc=False),
        scratch_types=[
            pltpu.VMEM((_NBUF, _CHUNK), jnp.int32),
            pltpu.VMEM((_NBUF, 3 * _CHUNK), jnp.float32),
            pltpu.VMEM((_NBUF, _CHUNK, _D), jnp.float32),
            pltpu.VMEM((_NBUF, 3 * _CHUNK), jnp.float32),
            pltpu.SemaphoreType.DMA,
            pltpu.SemaphoreType.DMA,
            pltpu.SemaphoreType.DMA,
            pltpu.SemaphoreType.DMA,
        ],
    )
    return grid_kernel(table, idx, bary)


def kernel(pix_to_face, bary_coords, face_verts_colors):
    B, H, W, K = pix_to_face.shape
    F, V, C = face_verts_colors.shape
    n = B * H * W * K
    hw = H * W * K
    idx = pix_to_face.reshape(n).astype(jnp.int32)
    bary = jnp.transpose(bary_coords, (0, 1, 4, 3, 2)).reshape(3 * n)
    table16 = _pad_call(jnp.transpose(face_verts_colors, (1, 2, 0)).reshape(F * V * C))
    out_flat = _texture_call(table16, idx, bary, n, hw, W * K)
    return out_flat.reshape(B, C, H, W)
